# baseline (speedup 1.0000x reference)
import jax
import jax.numpy as jnp
from jax import lax
from jax.experimental import pallas as pl
from jax.experimental.pallas import tpu as pltpu, tpu_sc as plsc

N = 10000
E = 320000
D = 128
NC = 2
NS = 16
NW = NC * NS
EPW = E // NW
K = 125
NCH = EPW // K
RA = 624
RL = N - (NS - 1) * RA
RC = 48

_f32 = jnp.float32


def _zero_vmem_2d(ref, rows, cols):
  z16 = jnp.zeros((16,), _f32)

  @pl.loop(0, rows * (cols // 16))
  def _(t):
    r = t // (cols // 16)
    j = t % (cols // 16)
    ref[r, pl.ds(j * 16, 16)] = z16


def _tile_rows(s, fn):
  base = pl.multiple_of(s * RA, 8)

  @pl.loop(0, RA // RC)
  def _(q):
    fn(pl.multiple_of(base + q * RC, 8), RC)

  @pl.when(s == NS - 1)
  def _():
    fn(N - (RL - RA), RL - RA)


def _deg_body(dst_hbm, out_hbm, degacc, idx_v, ones_v, zbuf,
              semI0, semI1, semI2, semI3, semS0, semS1):
  c = lax.axis_index("c")
  s = lax.axis_index("s")
  wid = s * NC + c
  semI = [semI0, semI1, semI2, semI3]
  semS = [semS0, semS1]

  def idx_load(g, sl):
    pltpu.async_copy(dst_hbm.at[wid, g], idx_v.at[sl, 1], semI[sl])

  def idx_wait(sl):
    pltpu.make_async_copy(dst_hbm.at[wid, 0], idx_v.at[sl, 1], semI[sl]
                          ).wait()

  def scat(sl, sp):
    pltpu.async_copy(ones_v, degacc.at[idx_v.at[sl, 1]], semS[sp], add=True)

  def swait(sp):
    pltpu.make_async_copy(ones_v, degacc.at[idx_v.at[0, 1]], semS[sp]).wait()

  for sl in range(4):
    idx_load(sl, sl)

  o16 = jnp.ones((16,), _f32)

  @pl.loop(0, K * (D // 16))
  def _(t):
    ones_v[t // (D // 16), pl.ds((t % (D // 16)) * 16, 16)] = o16

  _zero_vmem_2d(zbuf, RC, D)
  _tile_rows(s, lambda off, nr: pltpu.sync_copy(
      zbuf.at[pl.ds(0, nr)], degacc.at[pl.ds(off, nr)]))
  plsc.subcore_barrier()

  @pl.loop(0, NCH // 4)
  def _(i):
    for j in range(4):
      g = 4 * i + j
      idx_wait(j)
      scat(j, j % 2)

      @pl.when(g >= 1)
      def _():
        swait((j + 1) % 2)

      @pl.when(jnp.logical_and(g >= 1, g + 3 < NCH))
      def _():
        idx_load(g + 3, (j + 3) % 4)

  swait(1)
  plsc.subcore_barrier()

  def copy_out(off, nr):
    pltpu.sync_copy(degacc.at[pl.ds(off, nr)], out_hbm.at[c, pl.ds(off, nr)])

  _tile_rows(s, copy_out)


def _agg_body(src_hbm, dst_hbm, t_hbm, out_hbm, acc, idx_v, rows, zbuf,
              semI0, semI1, semI2, semI3, semR0, semR1, semS0, semS1):
  c = lax.axis_index("c")
  s = lax.axis_index("s")
  wid = s * NC + c
  semI = [semI0, semI1, semI2, semI3]
  semR = [semR0, semR1]
  semS = [semS0, semS1]

  def idx_load(g, sl):
    pltpu.async_copy(src_hbm.at[wid, g], idx_v.at[sl, 0], semI[sl])
    pltpu.async_copy(dst_hbm.at[wid, g], idx_v.at[sl, 1], semI[sl])

  def idx_wait(sl):
    pltpu.make_async_copy(src_hbm.at[wid, 0], idx_v.at[sl, 0], semI[sl]
                          ).wait()
    pltpu.make_async_copy(dst_hbm.at[wid, 0], idx_v.at[sl, 1], semI[sl]
                          ).wait()

  def gather(sl, rb):
    pltpu.async_copy(t_hbm.at[idx_v.at[sl, 0]], rows.at[rb], semR[rb])

  def gwait(rb):
    pltpu.make_async_copy(t_hbm.at[idx_v.at[0, 0]], rows.at[rb], semR[rb]
                          ).wait()

  def scat(sl, rb):
    pltpu.async_copy(rows.at[rb], acc.at[idx_v.at[sl, 1]], semS[rb],
                     add=True)

  def swait(sp):
    pltpu.make_async_copy(rows.at[0], acc.at[idx_v.at[0, 1]], semS[sp]
                          ).wait()

  for sl in range(4):
    idx_load(sl, sl)
  idx_wait(0)
  gather(0, 0)

  _zero_vmem_2d(zbuf, RC, D)
  _tile_rows(s, lambda off, nr: pltpu.sync_copy(
      zbuf.at[pl.ds(0, nr)], acc.at[pl.ds(off, nr)]))
  plsc.subcore_barrier()

  @pl.loop(0, NCH // 4)
  def _(i):
    for j in range(4):
      g = 4 * i + j
      nxt = (j + 1) % 4

      @pl.when(g >= 1)
      def _():
        swait((j + 1) % 2)

      @pl.when(g + 1 < NCH)
      def _():
        idx_wait(nxt)
        gather(nxt, (j + 1) % 2)

      gwait(j % 2)
      scat(j, j % 2)

      @pl.when(jnp.logical_and(g >= 2, g + 2 < NCH))
      def _():
        idx_load(g + 2, (j + 2) % 4)

  swait(1)
  plsc.subcore_barrier()

  def copy_out(off, nr):
    pltpu.sync_copy(acc.at[pl.ds(off, nr)], out_hbm.at[c, pl.ds(off, nr)])

  _tile_rows(s, copy_out)


def _sc_deg(dst3):
  mesh = plsc.VectorSubcoreMesh(core_axis_name="c", subcore_axis_name="s")
  f = pl.kernel(
      _deg_body,
      out_type=jax.ShapeDtypeStruct((NC, N, D), _f32),
      mesh=mesh,
      scratch_types=[
          pltpu.VMEM_SHARED((N, D), _f32),
          pltpu.VMEM((4, 2, K), jnp.int32),
          pltpu.VMEM((K, D), _f32),
          pltpu.VMEM((RC, D), _f32),
          pltpu.SemaphoreType.DMA,
          pltpu.SemaphoreType.DMA,
          pltpu.SemaphoreType.DMA,
          pltpu.SemaphoreType.DMA,
          pltpu.SemaphoreType.DMA,
          pltpu.SemaphoreType.DMA,
      ],
  )
  return f(dst3)


def _sc_aggregate(src3, dst3, t):
  mesh = plsc.VectorSubcoreMesh(core_axis_name="c", subcore_axis_name="s")
  f = pl.kernel(
      _agg_body,
      out_type=jax.ShapeDtypeStruct((NC, N, D), _f32),
      mesh=mesh,
      scratch_types=[
          pltpu.VMEM_SHARED((N, D), _f32),
          pltpu.VMEM((4, 2, K), jnp.int32),
          pltpu.VMEM((2, K, D), _f32),
          pltpu.VMEM((RC, D), _f32),
          pltpu.SemaphoreType.DMA,
          pltpu.SemaphoreType.DMA,
          pltpu.SemaphoreType.DMA,
          pltpu.SemaphoreType.DMA,
          pltpu.SemaphoreType.DMA,
          pltpu.SemaphoreType.DMA,
          pltpu.SemaphoreType.DMA,
          pltpu.SemaphoreType.DMA,
      ],
  )
  return f(src3, dst3, t)


BR = 1000
GRID = N // BR
_HI = jax.lax.Precision.HIGHEST


def _dinv_block(deg_ref):
  d = deg_ref[0, :, 0:1] + deg_ref[1, :, 0:1]
  return lax.rsqrt(d + 1.0)


def _k1_body(deg_ref, x_ref, w_ref, o_ref):
  dinv = _dinv_block(deg_ref)
  o_ref[:, :] = jnp.dot(x_ref[:, :], w_ref[:, :], precision=_HI) * dinv


def _k2_body(deg_ref, s_ref, t_ref, b_ref, w_ref, o_ref):
  dinv = _dinv_block(deg_ref)
  agg = s_ref[0] + s_ref[1] + t_ref[:, :]
  h = jnp.maximum(agg * dinv + b_ref[:, :], 0.0)
  o_ref[:, :] = jnp.dot(h, w_ref[:, :], precision=_HI) * dinv


def _k3_body(deg_ref, s_ref, t_ref, b_ref, o_ref):
  dinv = _dinv_block(deg_ref)
  o_ref[:, :] = (s_ref[0] + s_ref[1] + t_ref[:, :]) * dinv + b_ref[:, :]


_row_spec = pl.BlockSpec((BR, D), lambda i: (i, 0))
_par_spec = pl.BlockSpec((NC, BR, D), lambda i: (0, i, 0))
_mat_spec = pl.BlockSpec((D, D), lambda i: (0, 0))
_bias_spec = pl.BlockSpec((1, D), lambda i: (0, 0))
_out_t = jax.ShapeDtypeStruct((N, D), _f32)


def _tc_k1(degp, x, w1):
  return pl.pallas_call(
      _k1_body,
      grid=(GRID,),
      in_specs=[_par_spec, _row_spec, _mat_spec],
      out_specs=_row_spec,
      out_shape=_out_t,
  )(degp, x, w1)


def _tc_k2(degp, s1, t1, b1, w2):
  return pl.pallas_call(
      _k2_body,
      grid=(GRID,),
      in_specs=[_par_spec, _par_spec, _row_spec, _bias_spec, _mat_spec],
      out_specs=_row_spec,
      out_shape=_out_t,
  )(degp, s1, t1, b1, w2)


def _tc_k3(degp, s2, t2, b2):
  return pl.pallas_call(
      _k3_body,
      grid=(GRID,),
      in_specs=[_par_spec, _par_spec, _row_spec, _bias_spec],
      out_specs=_row_spec,
      out_shape=_out_t,
  )(degp, s2, t2, b2)


@jax.jit
def kernel(x, edge_index, W1, b1, W2, b2):
  src3 = edge_index[0].reshape(NW, NCH, K)
  dst3 = edge_index[1].reshape(NW, NCH, K)
  b1r = b1.reshape(1, D)
  b2r = b2.reshape(1, D)

  degp = _sc_deg(dst3)
  t1 = _tc_k1(degp, x, W1)
  s1 = _sc_aggregate(src3, dst3, t1)
  t2 = _tc_k2(degp, s1, t1, b1r, W2)
  s2 = _sc_aggregate(src3, dst3, t2)
  return _tc_k3(degp, s2, t2, b2r)


# speedup vs baseline: 1.0133x; 1.0133x over previous
import jax
import jax.numpy as jnp
from jax import lax
from jax.experimental import pallas as pl
from jax.experimental.pallas import tpu as pltpu, tpu_sc as plsc

N = 10000
E = 320000
D = 128
NC = 2
NS = 16
NW = NC * NS
EPW = E // NW
K = 125
NCH = EPW // K
RA = 624
RL = N - (NS - 1) * RA
RC = 48

_f32 = jnp.float32


def _zero_vmem_2d(ref, rows, cols):
  z16 = jnp.zeros((16,), _f32)

  @pl.loop(0, rows * (cols // 16))
  def _(t):
    r = t // (cols // 16)
    j = t % (cols // 16)
    ref[r, pl.ds(j * 16, 16)] = z16


def _tile_rows(s, fn):
  base = pl.multiple_of(s * RA, 8)

  @pl.loop(0, RA // RC)
  def _(q):
    fn(pl.multiple_of(base + q * RC, 8), RC)

  @pl.when(s == NS - 1)
  def _():
    fn(N - (RL - RA), RL - RA)


def _deg_body(dst_hbm, out_hbm, degacc, idx_v, ones_v, zbuf,
              semI0, semI1, semI2, semI3, semS0, semS1):
  c = lax.axis_index("c")
  s = lax.axis_index("s")
  wid = s * NC + c
  semI = [semI0, semI1, semI2, semI3]
  semS = [semS0, semS1]

  def idx_load(g, sl):
    pltpu.async_copy(dst_hbm.at[wid, g], idx_v.at[sl, 1], semI[sl])

  def idx_wait(sl):
    pltpu.make_async_copy(dst_hbm.at[wid, 0], idx_v.at[sl, 1], semI[sl]
                          ).wait()

  def scat(sl, sp):
    pltpu.async_copy(ones_v, degacc.at[idx_v.at[sl, 1]], semS[sp], add=True)

  def swait(sp):
    pltpu.make_async_copy(ones_v, degacc.at[idx_v.at[0, 1]], semS[sp]).wait()

  for sl in range(4):
    idx_load(sl, sl)

  o16 = jnp.ones((16,), _f32)

  @pl.loop(0, K * (D // 16))
  def _(t):
    ones_v[t // (D // 16), pl.ds((t % (D // 16)) * 16, 16)] = o16

  _zero_vmem_2d(zbuf, RC, D)
  _tile_rows(s, lambda off, nr: pltpu.sync_copy(
      zbuf.at[pl.ds(0, nr)], degacc.at[pl.ds(off, nr)]))
  plsc.subcore_barrier()

  @pl.loop(0, NCH // 4)
  def _(i):
    for j in range(4):
      g = 4 * i + j
      idx_wait(j)
      scat(j, j % 2)

      @pl.when(g >= 1)
      def _():
        swait((j + 1) % 2)

      @pl.when(jnp.logical_and(g >= 1, g + 3 < NCH))
      def _():
        idx_load(g + 3, (j + 3) % 4)

  swait(1)
  plsc.subcore_barrier()

  def copy_out(off, nr):
    pltpu.sync_copy(degacc.at[pl.ds(off, nr)], zbuf.at[pl.ds(0, nr)])
    pltpu.sync_copy(zbuf.at[pl.ds(0, nr)], out_hbm.at[c, pl.ds(off, nr)])

  _tile_rows(s, copy_out)


def _agg_body(src_hbm, dst_hbm, t_hbm, out_hbm, acc, idx_v, rows, zbuf,
              semI0, semI1, semI2, semI3, semR0, semR1, semS0, semS1):
  c = lax.axis_index("c")
  s = lax.axis_index("s")
  wid = s * NC + c
  semI = [semI0, semI1, semI2, semI3]
  semR = [semR0, semR1]
  semS = [semS0, semS1]

  def idx_load(g, sl):
    pltpu.async_copy(src_hbm.at[wid, g], idx_v.at[sl, 0], semI[sl])
    pltpu.async_copy(dst_hbm.at[wid, g], idx_v.at[sl, 1], semI[sl])

  def idx_wait(sl):
    pltpu.make_async_copy(src_hbm.at[wid, 0], idx_v.at[sl, 0], semI[sl]
                          ).wait()
    pltpu.make_async_copy(dst_hbm.at[wid, 0], idx_v.at[sl, 1], semI[sl]
                          ).wait()

  def gather(sl, rb):
    pltpu.async_copy(t_hbm.at[idx_v.at[sl, 0]], rows.at[rb], semR[rb])

  def gwait(rb):
    pltpu.make_async_copy(t_hbm.at[idx_v.at[0, 0]], rows.at[rb], semR[rb]
                          ).wait()

  def scat(sl, rb):
    pltpu.async_copy(rows.at[rb], acc.at[idx_v.at[sl, 1]], semS[rb],
                     add=True)

  def swait(sp):
    pltpu.make_async_copy(rows.at[0], acc.at[idx_v.at[0, 1]], semS[sp]
                          ).wait()

  for sl in range(4):
    idx_load(sl, sl)
  idx_wait(0)
  gather(0, 0)

  _zero_vmem_2d(zbuf, RC, D)
  _tile_rows(s, lambda off, nr: pltpu.sync_copy(
      zbuf.at[pl.ds(0, nr)], acc.at[pl.ds(off, nr)]))
  plsc.subcore_barrier()

  @pl.loop(0, NCH // 4)
  def _(i):
    for j in range(4):
      g = 4 * i + j
      nxt = (j + 1) % 4

      @pl.when(g >= 1)
      def _():
        swait((j + 1) % 2)

      @pl.when(g + 1 < NCH)
      def _():
        idx_wait(nxt)
        gather(nxt, (j + 1) % 2)

      gwait(j % 2)
      scat(j, j % 2)

      @pl.when(jnp.logical_and(g >= 2, g + 2 < NCH))
      def _():
        idx_load(g + 2, (j + 2) % 4)

  swait(1)
  plsc.subcore_barrier()

  def copy_out(off, nr):
    pltpu.sync_copy(acc.at[pl.ds(off, nr)], zbuf.at[pl.ds(0, nr)])
    pltpu.sync_copy(zbuf.at[pl.ds(0, nr)], out_hbm.at[c, pl.ds(off, nr)])

  _tile_rows(s, copy_out)


def _sc_deg(dst3):
  mesh = plsc.VectorSubcoreMesh(core_axis_name="c", subcore_axis_name="s")
  f = pl.kernel(
      _deg_body,
      out_type=jax.ShapeDtypeStruct((NC, N, D), _f32),
      mesh=mesh,
      scratch_types=[
          pltpu.VMEM_SHARED((N, D), _f32),
          pltpu.VMEM((4, 2, K), jnp.int32),
          pltpu.VMEM((K, D), _f32),
          pltpu.VMEM((RC, D), _f32),
          pltpu.SemaphoreType.DMA,
          pltpu.SemaphoreType.DMA,
          pltpu.SemaphoreType.DMA,
          pltpu.SemaphoreType.DMA,
          pltpu.SemaphoreType.DMA,
          pltpu.SemaphoreType.DMA,
      ],
  )
  return f(dst3)


def _sc_aggregate(src3, dst3, t):
  mesh = plsc.VectorSubcoreMesh(core_axis_name="c", subcore_axis_name="s")
  f = pl.kernel(
      _agg_body,
      out_type=jax.ShapeDtypeStruct((NC, N, D), _f32),
      mesh=mesh,
      scratch_types=[
          pltpu.VMEM_SHARED((N, D), _f32),
          pltpu.VMEM((4, 2, K), jnp.int32),
          pltpu.VMEM((2, K, D), _f32),
          pltpu.VMEM((RC, D), _f32),
          pltpu.SemaphoreType.DMA,
          pltpu.SemaphoreType.DMA,
          pltpu.SemaphoreType.DMA,
          pltpu.SemaphoreType.DMA,
          pltpu.SemaphoreType.DMA,
          pltpu.SemaphoreType.DMA,
          pltpu.SemaphoreType.DMA,
          pltpu.SemaphoreType.DMA,
      ],
  )
  return f(src3, dst3, t)


BR = 1000
GRID = N // BR
_HI = jax.lax.Precision.HIGHEST


def _dinv_block(deg_ref):
  d = deg_ref[0, :, 0:1] + deg_ref[1, :, 0:1]
  return lax.rsqrt(d + 1.0)


def _k1_body(deg_ref, x_ref, w_ref, o_ref):
  dinv = _dinv_block(deg_ref)
  o_ref[:, :] = jnp.dot(x_ref[:, :], w_ref[:, :], precision=_HI) * dinv


def _k2_body(deg_ref, s_ref, t_ref, b_ref, w_ref, o_ref):
  dinv = _dinv_block(deg_ref)
  agg = s_ref[0] + s_ref[1] + t_ref[:, :]
  h = jnp.maximum(agg * dinv + b_ref[:, :], 0.0)
  o_ref[:, :] = jnp.dot(h, w_ref[:, :], precision=_HI) * dinv


def _k3_body(deg_ref, s_ref, t_ref, b_ref, o_ref):
  dinv = _dinv_block(deg_ref)
  o_ref[:, :] = (s_ref[0] + s_ref[1] + t_ref[:, :]) * dinv + b_ref[:, :]


_row_spec = pl.BlockSpec((BR, D), lambda i: (i, 0))
_par_spec = pl.BlockSpec((NC, BR, D), lambda i: (0, i, 0))
_mat_spec = pl.BlockSpec((D, D), lambda i: (0, 0))
_bias_spec = pl.BlockSpec((1, D), lambda i: (0, 0))
_out_t = jax.ShapeDtypeStruct((N, D), _f32)


def _tc_k1(degp, x, w1):
  return pl.pallas_call(
      _k1_body,
      grid=(GRID,),
      in_specs=[_par_spec, _row_spec, _mat_spec],
      out_specs=_row_spec,
      out_shape=_out_t,
  )(degp, x, w1)


def _tc_k2(degp, s1, t1, b1, w2):
  return pl.pallas_call(
      _k2_body,
      grid=(GRID,),
      in_specs=[_par_spec, _par_spec, _row_spec, _bias_spec, _mat_spec],
      out_specs=_row_spec,
      out_shape=_out_t,
  )(degp, s1, t1, b1, w2)


def _tc_k3(degp, s2, t2, b2):
  return pl.pallas_call(
      _k3_body,
      grid=(GRID,),
      in_specs=[_par_spec, _par_spec, _row_spec, _bias_spec],
      out_specs=_row_spec,
      out_shape=_out_t,
  )(degp, s2, t2, b2)


@jax.jit
def kernel(x, edge_index, W1, b1, W2, b2):
  src3 = edge_index[0].reshape(NW, NCH, K)
  dst3 = edge_index[1].reshape(NW, NCH, K)
  b1r = b1.reshape(1, D)
  b2r = b2.reshape(1, D)

  degp = _sc_deg(dst3)
  t1 = _tc_k1(degp, x, W1)
  s1 = _sc_aggregate(src3, dst3, t1)
  t2 = _tc_k2(degp, s1, t1, b1r, W2)
  s2 = _sc_aggregate(src3, dst3, t2)
  return _tc_k3(degp, s2, t2, b2r)


# speedup vs baseline: 1.0425x; 1.0288x over previous
import jax
import jax.numpy as jnp
from jax import lax
from jax.experimental import pallas as pl
from jax.experimental.pallas import tpu as pltpu, tpu_sc as plsc

N = 10000
E = 320000
D = 128
NC = 2
NS = 16
NW = NC * NS
EPW = E // NW
K = 125
NCH = EPW // K
RA = 624
RL = N - (NS - 1) * RA
RC = 48

_f32 = jnp.float32


def _off_q(s, q):
  return pl.multiple_of(pl.multiple_of(s * RA, 8) + q * RC, 8)


def _piped_zero(s, zbuf, acc, sem):
  z16 = jnp.zeros((16,), _f32)

  @pl.loop(0, RC * (D // 16))
  def _(t):
    zbuf[0, t // (D // 16), pl.ds((t % (D // 16)) * 16, 16)] = z16

  for q in range(RA // RC):
    pltpu.async_copy(zbuf.at[0], acc.at[pl.ds(_off_q(s, q), RC)], sem)

  @pl.when(s == NS - 1)
  def _():
    pltpu.async_copy(zbuf.at[0, pl.ds(0, RL - RA)],
                     acc.at[pl.ds(N - (RL - RA), RL - RA)], sem)

  for q in range(RA // RC):
    pltpu.make_async_copy(zbuf.at[0], acc.at[pl.ds(0, RC)], sem).wait()

  @pl.when(s == NS - 1)
  def _():
    pltpu.make_async_copy(zbuf.at[0, pl.ds(0, RL - RA)],
                          acc.at[pl.ds(0, RL - RA)], sem).wait()


def _piped_copy_out(c, s, acc, zbuf, out_hbm, semA, semB):
  nq = RA // RC

  def hop1(q, p):
    pltpu.async_copy(acc.at[pl.ds(_off_q(s, q), RC)], zbuf.at[p], semA[p])

  def hop2(q, p):
    pltpu.async_copy(zbuf.at[p], out_hbm.at[c, pl.ds(_off_q(s, q), RC)],
                     semB[p])

  def wait1(p):
    pltpu.make_async_copy(acc.at[pl.ds(0, RC)], zbuf.at[p], semA[p]).wait()

  def wait2(p):
    pltpu.make_async_copy(zbuf.at[p], out_hbm.at[c, pl.ds(0, RC)], semB[p]
                          ).wait()

  hop1(0, 0)
  for q in range(1, nq):
    p = q % 2
    wait1(1 - p)
    hop2(q - 1, 1 - p)
    if q >= 2:
      wait2(p)
    hop1(q, p)
  wait1((nq - 1) % 2)
  hop2(nq - 1, (nq - 1) % 2)
  wait2(nq % 2)
  wait2((nq - 1) % 2)

  @pl.when(s == NS - 1)
  def _():
    tail = RL - RA
    pltpu.sync_copy(acc.at[pl.ds(N - tail, tail)],
                    zbuf.at[0, pl.ds(0, tail)])
    pltpu.sync_copy(zbuf.at[0, pl.ds(0, tail)],
                    out_hbm.at[c, pl.ds(N - tail, tail)])


def _deg_body(dst_hbm, out_hbm, degacc, idx_v, ones_v, zbuf,
              semI0, semI1, semI2, semI3, semS0, semS1):
  c = lax.axis_index("c")
  s = lax.axis_index("s")
  wid = s * NC + c
  semI = [semI0, semI1, semI2, semI3]
  semS = [semS0, semS1]

  def idx_load(g, sl):
    pltpu.async_copy(dst_hbm.at[wid, g], idx_v.at[sl, 1], semI[sl])

  def idx_wait(sl):
    pltpu.make_async_copy(dst_hbm.at[wid, 0], idx_v.at[sl, 1], semI[sl]
                          ).wait()

  def scat(sl, sp):
    pltpu.async_copy(ones_v, degacc.at[idx_v.at[sl, 1]], semS[sp], add=True)

  def swait(sp):
    pltpu.make_async_copy(ones_v, degacc.at[idx_v.at[0, 1]], semS[sp]).wait()

  for sl in range(4):
    idx_load(sl, sl)

  o16 = jnp.ones((16,), _f32)

  @pl.loop(0, K * (D // 16))
  def _(t):
    ones_v[t // (D // 16), pl.ds((t % (D // 16)) * 16, 16)] = o16

  _piped_zero(s, zbuf, degacc, semS0)
  plsc.subcore_barrier()

  @pl.loop(0, NCH // 4)
  def _(i):
    for j in range(4):
      g = 4 * i + j
      idx_wait(j)
      scat(j, j % 2)

      @pl.when(g >= 1)
      def _():
        swait((j + 1) % 2)

      @pl.when(jnp.logical_and(g >= 1, g + 3 < NCH))
      def _():
        idx_load(g + 3, (j + 3) % 4)

  swait(1)
  plsc.subcore_barrier()

  _piped_copy_out(c, s, degacc, zbuf, out_hbm, [semI0, semI1],
                  [semS0, semS1])


def _agg_body(src_hbm, dst_hbm, t_hbm, out_hbm, acc, idx_v, rows, zbuf,
              semI0, semI1, semI2, semI3, semR0, semR1, semS0, semS1):
  c = lax.axis_index("c")
  s = lax.axis_index("s")
  wid = s * NC + c
  semI = [semI0, semI1, semI2, semI3]
  semR = [semR0, semR1]
  semS = [semS0, semS1]

  def idx_load(g, sl):
    pltpu.async_copy(src_hbm.at[wid, g], idx_v.at[sl, 0], semI[sl])
    pltpu.async_copy(dst_hbm.at[wid, g], idx_v.at[sl, 1], semI[sl])

  def idx_wait(sl):
    pltpu.make_async_copy(src_hbm.at[wid, 0], idx_v.at[sl, 0], semI[sl]
                          ).wait()
    pltpu.make_async_copy(dst_hbm.at[wid, 0], idx_v.at[sl, 1], semI[sl]
                          ).wait()

  def gather(sl, rb):
    pltpu.async_copy(t_hbm.at[idx_v.at[sl, 0]], rows.at[rb], semR[rb])

  def gwait(rb):
    pltpu.make_async_copy(t_hbm.at[idx_v.at[0, 0]], rows.at[rb], semR[rb]
                          ).wait()

  def scat(sl, rb):
    pltpu.async_copy(rows.at[rb], acc.at[idx_v.at[sl, 1]], semS[rb],
                     add=True)

  def swait(sp):
    pltpu.make_async_copy(rows.at[0], acc.at[idx_v.at[0, 1]], semS[sp]
                          ).wait()

  for sl in range(4):
    idx_load(sl, sl)
  idx_wait(0)
  gather(0, 0)

  _piped_zero(s, zbuf, acc, semS0)
  plsc.subcore_barrier()

  @pl.loop(0, NCH // 4)
  def _(i):
    for j in range(4):
      g = 4 * i + j
      nxt = (j + 1) % 4

      @pl.when(g >= 1)
      def _():
        swait((j + 1) % 2)

      @pl.when(g + 1 < NCH)
      def _():
        idx_wait(nxt)
        gather(nxt, (j + 1) % 2)

      gwait(j % 2)
      scat(j, j % 2)

      @pl.when(jnp.logical_and(g >= 2, g + 2 < NCH))
      def _():
        idx_load(g + 2, (j + 2) % 4)

  swait(1)
  plsc.subcore_barrier()

  _piped_copy_out(c, s, acc, zbuf, out_hbm, [semR0, semR1], [semS0, semS1])


def _sc_deg(dst3):
  mesh = plsc.VectorSubcoreMesh(core_axis_name="c", subcore_axis_name="s")
  f = pl.kernel(
      _deg_body,
      out_type=jax.ShapeDtypeStruct((NC, N, D), _f32),
      mesh=mesh,
      scratch_types=[
          pltpu.VMEM_SHARED((N, D), _f32),
          pltpu.VMEM((4, 2, K), jnp.int32),
          pltpu.VMEM((K, D), _f32),
          pltpu.VMEM((2, RC, D), _f32),
          pltpu.SemaphoreType.DMA,
          pltpu.SemaphoreType.DMA,
          pltpu.SemaphoreType.DMA,
          pltpu.SemaphoreType.DMA,
          pltpu.SemaphoreType.DMA,
          pltpu.SemaphoreType.DMA,
      ],
  )
  return f(dst3)


def _sc_aggregate(src3, dst3, t):
  mesh = plsc.VectorSubcoreMesh(core_axis_name="c", subcore_axis_name="s")
  f = pl.kernel(
      _agg_body,
      out_type=jax.ShapeDtypeStruct((NC, N, D), _f32),
      mesh=mesh,
      scratch_types=[
          pltpu.VMEM_SHARED((N, D), _f32),
          pltpu.VMEM((4, 2, K), jnp.int32),
          pltpu.VMEM((2, K, D), _f32),
          pltpu.VMEM((2, RC, D), _f32),
          pltpu.SemaphoreType.DMA,
          pltpu.SemaphoreType.DMA,
          pltpu.SemaphoreType.DMA,
          pltpu.SemaphoreType.DMA,
          pltpu.SemaphoreType.DMA,
          pltpu.SemaphoreType.DMA,
          pltpu.SemaphoreType.DMA,
          pltpu.SemaphoreType.DMA,
      ],
  )
  return f(src3, dst3, t)


BR = 1000
GRID = N // BR
_HI = jax.lax.Precision.HIGHEST


def _dinv_block(deg_ref):
  d = deg_ref[0, :, 0:1] + deg_ref[1, :, 0:1]
  return lax.rsqrt(d + 1.0)


def _k1_body(deg_ref, x_ref, w_ref, o_ref):
  dinv = _dinv_block(deg_ref)
  o_ref[:, :] = jnp.dot(x_ref[:, :], w_ref[:, :], precision=_HI) * dinv


def _k2_body(deg_ref, s_ref, t_ref, b_ref, w_ref, o_ref):
  dinv = _dinv_block(deg_ref)
  agg = s_ref[0] + s_ref[1] + t_ref[:, :]
  h = jnp.maximum(agg * dinv + b_ref[:, :], 0.0)
  o_ref[:, :] = jnp.dot(h, w_ref[:, :], precision=_HI) * dinv


def _k3_body(deg_ref, s_ref, t_ref, b_ref, o_ref):
  dinv = _dinv_block(deg_ref)
  o_ref[:, :] = (s_ref[0] + s_ref[1] + t_ref[:, :]) * dinv + b_ref[:, :]


_row_spec = pl.BlockSpec((BR, D), lambda i: (i, 0))
_par_spec = pl.BlockSpec((NC, BR, D), lambda i: (0, i, 0))
_mat_spec = pl.BlockSpec((D, D), lambda i: (0, 0))
_bias_spec = pl.BlockSpec((1, D), lambda i: (0, 0))
_out_t = jax.ShapeDtypeStruct((N, D), _f32)


def _tc_k1(degp, x, w1):
  return pl.pallas_call(
      _k1_body,
      grid=(GRID,),
      in_specs=[_par_spec, _row_spec, _mat_spec],
      out_specs=_row_spec,
      out_shape=_out_t,
  )(degp, x, w1)


def _tc_k2(degp, s1, t1, b1, w2):
  return pl.pallas_call(
      _k2_body,
      grid=(GRID,),
      in_specs=[_par_spec, _par_spec, _row_spec, _bias_spec, _mat_spec],
      out_specs=_row_spec,
      out_shape=_out_t,
  )(degp, s1, t1, b1, w2)


def _tc_k3(degp, s2, t2, b2):
  return pl.pallas_call(
      _k3_body,
      grid=(GRID,),
      in_specs=[_par_spec, _par_spec, _row_spec, _bias_spec],
      out_specs=_row_spec,
      out_shape=_out_t,
  )(degp, s2, t2, b2)


@jax.jit
def kernel(x, edge_index, W1, b1, W2, b2):
  src3 = edge_index[0].reshape(NW, NCH, K)
  dst3 = edge_index[1].reshape(NW, NCH, K)
  b1r = b1.reshape(1, D)
  b2r = b2.reshape(1, D)

  degp = _sc_deg(dst3)
  t1 = _tc_k1(degp, x, W1)
  s1 = _sc_aggregate(src3, dst3, t1)
  t2 = _tc_k2(degp, s1, t1, b1r, W2)
  s2 = _sc_aggregate(src3, dst3, t2)
  return _tc_k3(degp, s2, t2, b2r)
